# baseline (speedup 1.0000x reference)
import jax
import jax.numpy as jnp
from jax.experimental import pallas as pl

_BLKN = 8192


def _body(gu_ref, fi_ref, out_ref):
    p = gu_ref[...] * fi_ref[...]
    out_ref[...] = jnp.sum(p, axis=0)


def kernel(gu, fi):
    B, D = gu.shape
    grid = pl.cdiv(B, _BLKN)
    out = pl.pallas_call(
        _body,
        grid=(grid,),
        in_specs=[
            pl.BlockSpec((D, _BLKN), lambda i: (0, i)),
            pl.BlockSpec((D, _BLKN), lambda i: (0, i)),
        ],
        out_specs=pl.BlockSpec((_BLKN,), lambda i: (i,)),
        out_shape=jax.ShapeDtypeStruct((B,), jnp.float32),
    )(gu.T, fi.T)
    return out


# speedup vs baseline: 1.0988x; 1.0988x over previous
import jax
import jax.numpy as jnp
from jax.experimental import pallas as pl

_BLKN = 20480


def _body(gu_ref, fi_ref, out_ref):
    p = gu_ref[...] * fi_ref[...]
    out_ref[...] = jnp.sum(p, axis=0)


def kernel(gu, fi):
    B, D = gu.shape
    grid = pl.cdiv(B, _BLKN)
    out = pl.pallas_call(
        _body,
        grid=(grid,),
        in_specs=[
            pl.BlockSpec((D, _BLKN), lambda i: (0, i)),
            pl.BlockSpec((D, _BLKN), lambda i: (0, i)),
        ],
        out_specs=pl.BlockSpec((_BLKN,), lambda i: (i,)),
        out_shape=jax.ShapeDtypeStruct((B,), jnp.float32),
    )(gu.T, fi.T)
    return out


# speedup vs baseline: 1.1036x; 1.0043x over previous
import jax
import jax.numpy as jnp
from jax.experimental import pallas as pl

_BLKN = 16384


def _body(gu_ref, fi_ref, out_ref):
    p = gu_ref[...] * fi_ref[...]
    out_ref[...] = jnp.sum(p, axis=0)


def kernel(gu, fi):
    B, D = gu.shape
    grid = pl.cdiv(B, _BLKN)
    out = pl.pallas_call(
        _body,
        grid=(grid,),
        in_specs=[
            pl.BlockSpec((D, _BLKN), lambda i: (0, i)),
            pl.BlockSpec((D, _BLKN), lambda i: (0, i)),
        ],
        out_specs=pl.BlockSpec((_BLKN,), lambda i: (i,)),
        out_shape=jax.ShapeDtypeStruct((B,), jnp.float32),
    )(gu.T, fi.T)
    return out
